# baseline (speedup 1.0000x reference)
import functools

import jax
import jax.numpy as jnp
from jax import lax
from jax.experimental import pallas as pl
from jax.experimental.pallas import tpu as pltpu
from jax.experimental.pallas import tpu_sc as plsc

B = 16384
E = 32
F = 32
H = 256
PACK = 128 // E
NPACK = 1_000_000 // PACK

_info = plsc.get_sparse_core_info()
_NC, _NS = _info.num_cores, _info.num_subcores
_NW = _NC * _NS
_BPW = B // _NW
_NCH = 2
_CH = _BPW // _NCH

_sc_mesh = plsc.VectorSubcoreMesh(core_axis_name="c", subcore_axis_name="s")


@functools.partial(
    pl.kernel,
    out_type=(
        jax.ShapeDtypeStruct((B, 128), jnp.float32),
        jax.ShapeDtypeStruct((B, 128), jnp.float32),
    ),
    mesh=_sc_mesh,
    scratch_types=[
        [pltpu.VMEM((_CH,), jnp.int32) for _ in range(_NCH)],
        pltpu.VMEM((_CH, 128), jnp.float32),
        [pltpu.VMEM((_CH,), jnp.int32) for _ in range(_NCH)],
        pltpu.VMEM((_CH, 128), jnp.float32),
        pltpu.SemaphoreType.DMA,
        pltpu.SemaphoreType.DMA,
    ],
)
def _gather_sc(uids_hbm, utab_hbm, rids_hbm, rtab_hbm, uout_hbm, rout_hbm,
               uidx_v, urows_v, ridx_v, rrows_v, usem, rsem):
    wid = lax.axis_index("s") * _NC + lax.axis_index("c")
    base = wid * _BPW
    for c in range(_NCH):
        pltpu.sync_copy(uids_hbm.at[pl.ds(base + c * _CH, _CH)], uidx_v[c])
        pltpu.sync_copy(rids_hbm.at[pl.ds(base + c * _CH, _CH)], ridx_v[c])
        ucp = pltpu.async_copy(utab_hbm.at[uidx_v[c]], urows_v, usem)
        rcp = pltpu.async_copy(rtab_hbm.at[ridx_v[c]], rrows_v, rsem)
        ucp.wait()
        rcp.wait()
        pltpu.sync_copy(urows_v, uout_hbm.at[pl.ds(base + c * _CH, _CH)])
        pltpu.sync_copy(rrows_v, rout_hbm.at[pl.ds(base + c * _CH, _CH)])


_BN = 2048


def _towers_body(urows, um, ufeat, rrows, rm, rfeat,
                 uW1t, ub1, uW2t, ub2, uW3t, ub3,
                 rW1t, rb1, rW2t, rb2, rW3t, rb3, out):
    def pick(rows, m):
        emb = jnp.zeros((_BN, E), jnp.float32)
        for k in range(PACK):
            emb = emb + jnp.where(m[...] == k, rows[:, k * E:(k + 1) * E], 0.0)
        return emb

    def tower(emb, feat, W1t, b1, W2t, b2, W3t, b3):
        x = jnp.concatenate([emb, feat[...]], axis=1)
        h = jnp.dot(x, W1t[...], preferred_element_type=jnp.float32) + b1[...]
        h = jnp.maximum(h, 0.0)
        h = jnp.dot(h, W2t[...], preferred_element_type=jnp.float32) + b2[...]
        h = jnp.maximum(h, 0.0)
        o = jnp.dot(h, W3t[...], preferred_element_type=jnp.float32) + b3[...]
        n = jnp.sqrt(jnp.sum(o * o, axis=1, keepdims=True))
        return o / jnp.maximum(n, 1e-12)

    u = tower(pick(urows, um), ufeat, uW1t, ub1, uW2t, ub2, uW3t, ub3)
    r = tower(pick(rrows, rm), rfeat, rW1t, rb1, rW2t, rb2, rW3t, rb3)
    out[...] = jnp.sum(u * r, axis=1, keepdims=True)


def _full(shape):
    return pl.BlockSpec(shape, lambda i: (0,) * len(shape))


_towers_tc = pl.pallas_call(
    _towers_body,
    grid=(B // _BN,),
    in_specs=[
        pl.BlockSpec((_BN, 128), lambda i: (i, 0)),
        pl.BlockSpec((_BN, 1), lambda i: (i, 0)),
        pl.BlockSpec((_BN, F), lambda i: (i, 0)),
        pl.BlockSpec((_BN, 128), lambda i: (i, 0)),
        pl.BlockSpec((_BN, 1), lambda i: (i, 0)),
        pl.BlockSpec((_BN, F), lambda i: (i, 0)),
        _full((E + F, H)), _full((1, H)),
        _full((H, H)), _full((1, H)),
        _full((H, E)), _full((1, E)),
        _full((E + F, H)), _full((1, H)),
        _full((H, H)), _full((1, H)),
        _full((H, E)), _full((1, E)),
    ],
    out_specs=pl.BlockSpec((_BN, 1), lambda i: (i, 0)),
    out_shape=jax.ShapeDtypeStruct((B, 1), jnp.float32),
)


def kernel(user_ids, user_features, recipe_ids, recipe_features,
           user_table, recipe_table,
           uW1, ub1, uW2, ub2, uW3, ub3,
           rW1, rb1, rW2, rb2, rW3, rb3):
    uids = user_ids.astype(jnp.int32)
    rids = recipe_ids.astype(jnp.int32)
    utab4 = user_table.reshape(NPACK, 128)
    rtab4 = recipe_table.reshape(NPACK, 128)
    urows, rrows = _gather_sc(uids // PACK, utab4, rids // PACK, rtab4)
    um = (uids % PACK).reshape(B, 1)
    rm = (rids % PACK).reshape(B, 1)
    scores = _towers_tc(
        urows, um, user_features, rrows, rm, recipe_features,
        uW1.T, ub1.reshape(1, H), uW2.T, ub2.reshape(1, H),
        uW3.T, ub3.reshape(1, E),
        rW1.T, rb1.reshape(1, H), rW2.T, rb2.reshape(1, H),
        rW3.T, rb3.reshape(1, E),
    )
    return scores.reshape(B)


# speedup vs baseline: 1.5426x; 1.5426x over previous
import functools

import jax
import jax.numpy as jnp
from jax import lax
from jax.experimental import pallas as pl
from jax.experimental.pallas import tpu as pltpu
from jax.experimental.pallas import tpu_sc as plsc

B = 16384
E = 32
F = 32
H = 256

_info = plsc.get_sparse_core_info()
_NC, _NS = _info.num_cores, _info.num_subcores
_NW = _NC * _NS
_BPW = B // _NW
_NCH = 2
_CH = _BPW // _NCH

_sc_mesh = plsc.VectorSubcoreMesh(core_axis_name="c", subcore_axis_name="s")


@functools.partial(
    pl.kernel,
    out_type=(
        jax.ShapeDtypeStruct((B, E), jnp.float32),
        jax.ShapeDtypeStruct((B, E), jnp.float32),
    ),
    mesh=_sc_mesh,
    compiler_params=pltpu.CompilerParams(needs_layout_passes=False),
    scratch_types=[
        pltpu.VMEM((_BPW,), jnp.int32),
        pltpu.VMEM((_CH, E), jnp.float32),
        pltpu.VMEM((_BPW,), jnp.int32),
        pltpu.VMEM((_CH, E), jnp.float32),
        pltpu.SemaphoreType.DMA,
        pltpu.SemaphoreType.DMA,
    ],
)
def _gather_sc(uids_hbm, utab_hbm, rids_hbm, rtab_hbm, uout_hbm, rout_hbm,
               uidx_v, urows_v, ridx_v, rrows_v, usem, rsem):
    wid = lax.axis_index("s") * _NC + lax.axis_index("c")
    base = wid * _BPW
    lanes = lax.iota(jnp.int32, 16)
    pltpu.sync_copy(uids_hbm.at[pl.ds(base, _BPW)], uidx_v)
    pltpu.sync_copy(rids_hbm.at[pl.ds(base, _BPW)], ridx_v)

    for c in range(_NCH):
        def body(g, carry):
            uvec = uidx_v[pl.ds((c * _CH + g * 16), 16)]
            rvec = ridx_v[pl.ds((c * _CH + g * 16), 16)]
            for j in range(16):
                uidx = jnp.sum(jnp.where(lanes == j, uvec, 0))
                ridx = jnp.sum(jnp.where(lanes == j, rvec, 0))
                pltpu.async_copy(utab_hbm.at[pl.ds(uidx, 1), :],
                                 urows_v.at[pl.ds(g * 16 + j, 1), :], usem)
                pltpu.async_copy(rtab_hbm.at[pl.ds(ridx, 1), :],
                                 rrows_v.at[pl.ds(g * 16 + j, 1), :], rsem)
            return carry

        lax.fori_loop(0, _CH // 16, body, 0)
        pltpu.make_async_copy(uout_hbm.at[pl.ds(0, _CH)], urows_v, usem).wait()
        pltpu.make_async_copy(rout_hbm.at[pl.ds(0, _CH)], rrows_v, rsem).wait()
        pltpu.sync_copy(urows_v, uout_hbm.at[pl.ds(base + c * _CH, _CH)])
        pltpu.sync_copy(rrows_v, rout_hbm.at[pl.ds(base + c * _CH, _CH)])


_BN = 2048


def _towers_body(uemb, ufeat, remb, rfeat,
                 uW1t, ub1, uW2t, ub2, uW3t, ub3,
                 rW1t, rb1, rW2t, rb2, rW3t, rb3, out):
    def tower(emb, feat, W1t, b1, W2t, b2, W3t, b3):
        x = jnp.concatenate([emb[...], feat[...]], axis=1)
        h = jnp.dot(x, W1t[...], preferred_element_type=jnp.float32) + b1[...]
        h = jnp.maximum(h, 0.0)
        h = jnp.dot(h, W2t[...], preferred_element_type=jnp.float32) + b2[...]
        h = jnp.maximum(h, 0.0)
        o = jnp.dot(h, W3t[...], preferred_element_type=jnp.float32) + b3[...]
        n = jnp.sqrt(jnp.sum(o * o, axis=1, keepdims=True))
        return o / jnp.maximum(n, 1e-12)

    u = tower(uemb, ufeat, uW1t, ub1, uW2t, ub2, uW3t, ub3)
    r = tower(remb, rfeat, rW1t, rb1, rW2t, rb2, rW3t, rb3)
    out[...] = jnp.sum(u * r, axis=1, keepdims=True)


def _full(shape):
    return pl.BlockSpec(shape, lambda i: (0,) * len(shape))


_towers_tc = pl.pallas_call(
    _towers_body,
    grid=(B // _BN,),
    in_specs=[
        pl.BlockSpec((_BN, E), lambda i: (i, 0)),
        pl.BlockSpec((_BN, F), lambda i: (i, 0)),
        pl.BlockSpec((_BN, E), lambda i: (i, 0)),
        pl.BlockSpec((_BN, F), lambda i: (i, 0)),
        _full((E + F, H)), _full((1, H)),
        _full((H, H)), _full((1, H)),
        _full((H, E)), _full((1, E)),
        _full((E + F, H)), _full((1, H)),
        _full((H, H)), _full((1, H)),
        _full((H, E)), _full((1, E)),
    ],
    out_specs=pl.BlockSpec((_BN, 1), lambda i: (i, 0)),
    out_shape=jax.ShapeDtypeStruct((B, 1), jnp.float32),
)


def kernel(user_ids, user_features, recipe_ids, recipe_features,
           user_table, recipe_table,
           uW1, ub1, uW2, ub2, uW3, ub3,
           rW1, rb1, rW2, rb2, rW3, rb3):
    uids = user_ids.astype(jnp.int32)
    rids = recipe_ids.astype(jnp.int32)
    uemb, remb = _gather_sc(uids, user_table, rids, recipe_table)
    scores = _towers_tc(
        uemb, user_features, remb, recipe_features,
        uW1.T, ub1.reshape(1, H), uW2.T, ub2.reshape(1, H),
        uW3.T, ub3.reshape(1, E),
        rW1.T, rb1.reshape(1, H), rW2.T, rb2.reshape(1, H),
        rW3.T, rb3.reshape(1, E),
    )
    return scores.reshape(B)
